# baseline (speedup 1.0000x reference)
import functools

import jax
import jax.numpy as jnp
from jax import lax
from jax.experimental import pallas as pl
from jax.experimental.pallas import tpu as pltpu
from jax.experimental.pallas import tpu_sc as plsc

_HPAD = 32


def _scores_block(x_ref, s_ref, *, h: int, w: int):
    xb = x_ref[...]
    act = jnp.sum(xb * xb, axis=2)
    lane = lax.broadcasted_iota(jnp.int32, (h, h * w), 1)
    row = lax.broadcasted_iota(jnp.int32, (h, h * w), 0)
    seg = (lane // w) == row
    neg = jnp.float32(-jnp.inf)
    scores = jnp.max(jnp.where(seg[None], act[:, None, :], neg), axis=2)
    bsz = scores.shape[0]
    pad = jnp.full((bsz, _HPAD - h), neg, scores.dtype)
    s_ref[...] = jnp.concatenate([scores, pad], axis=1)


def _apply_block(x_ref, m_ref, o_ref, *, h: int, w: int):
    xb = x_ref[...]
    keep = m_ref[...][:, :h]
    lane = lax.broadcasted_iota(jnp.int32, (h, h * w), 1)
    row = lax.broadcasted_iota(jnp.int32, (h, h * w), 0)
    seg = (lane // w) == row
    wide = jnp.sum(
        jnp.where(seg[None], keep[:, :, None], jnp.float32(0.0)), axis=1
    )
    o_ref[...] = xb * wide[:, :, None]


def _make_sc_mask(b: int, h: int, rh: int):
    mesh = plsc.VectorSubcoreMesh(core_axis_name="c", subcore_axis_name="s")
    n_workers = 32
    per_w = b // n_workers

    @functools.partial(
        pl.kernel,
        mesh=mesh,
        out_type=jax.ShapeDtypeStruct((b, _HPAD), jnp.float32),
        scratch_types=[
            pltpu.VMEM((per_w, _HPAD), jnp.float32),
            pltpu.VMEM((per_w, _HPAD), jnp.float32),
        ],
        compiler_params=pltpu.CompilerParams(needs_layout_passes=False),
    )
    def sc_mask(scores_hbm, mask_hbm, sc_v, mask_v):
        wid = lax.axis_index("s") * 2 + lax.axis_index("c")
        base = wid * per_w
        pltpu.sync_copy(scores_hbm.at[pl.ds(base, per_w)], sc_v)
        iota = lax.iota(jnp.int32, 16)
        for t in range(per_w):
            a = sc_v[t, pl.ds(0, 16)]
            bb = sc_v[t, pl.ds(16, 16)]
            keep_a = jnp.ones((16,), jnp.float32)
            keep_b = jnp.ones((16,), jnp.float32)
            for hh in range(h):
                if hh < 16:
                    sh = jnp.broadcast_to(a[hh], (16,))
                else:
                    sh = jnp.broadcast_to(bb[hh - 16], (16,))
                cnt = (plsc.all_reduce_population_count(a > sh)
                       + plsc.all_reduce_population_count(bb > sh))
                dropped = cnt < rh
                if hh < 16:
                    keep_a = jnp.where((iota == hh) & dropped,
                                       jnp.float32(0.0), keep_a)
                else:
                    keep_b = jnp.where((iota == hh - 16) & dropped,
                                       jnp.float32(0.0), keep_b)
            mask_v[t, pl.ds(0, 16)] = keep_a
            mask_v[t, pl.ds(16, 16)] = keep_b
        pltpu.sync_copy(mask_v, mask_hbm.at[pl.ds(base, per_w)])

    return sc_mask


@jax.jit
def kernel(x):
    b, c, h, w = x.shape
    rh = int(round(0.33 * h))
    xt = jnp.transpose(x, (0, 2, 3, 1)).reshape(b, h * w, c)
    b_blk = 8

    scores = pl.pallas_call(
        functools.partial(_scores_block, h=h, w=w),
        grid=(b // b_blk,),
        in_specs=[pl.BlockSpec((b_blk, h * w, c), lambda i: (i, 0, 0))],
        out_specs=pl.BlockSpec((b_blk, _HPAD), lambda i: (i, 0)),
        out_shape=jax.ShapeDtypeStruct((b, _HPAD), x.dtype),
    )(xt)

    mask = _make_sc_mask(b, h, rh)(scores)

    out = pl.pallas_call(
        functools.partial(_apply_block, h=h, w=w),
        grid=(b // b_blk,),
        in_specs=[
            pl.BlockSpec((b_blk, h * w, c), lambda i: (i, 0, 0)),
            pl.BlockSpec((b_blk, _HPAD), lambda i: (i, 0)),
        ],
        out_specs=pl.BlockSpec((b_blk, h * w, c), lambda i: (i, 0, 0)),
        out_shape=jax.ShapeDtypeStruct((b, h * w, c), x.dtype),
    )(xt, mask)

    return jnp.transpose(out.reshape(b, h, w, c), (0, 3, 1, 2))


# speedup vs baseline: 1.7494x; 1.7494x over previous
import functools

import jax
import jax.numpy as jnp
from jax import lax
from jax.experimental import pallas as pl


def _topdrop_block(x_ref, o_ref, *, h: int, w: int, rh: int):
    xb = x_ref[...]
    act = jnp.sum(xb * xb, axis=2)

    lane = lax.broadcasted_iota(jnp.int32, (h, h * w), 1)
    row = lax.broadcasted_iota(jnp.int32, (h, h * w), 0)
    seg = (lane // w) == row

    neg = jnp.float32(-jnp.inf)
    scores = jnp.max(
        jnp.where(seg[None], act[:, None, :], neg), axis=2
    )

    gt = (scores[:, None, :] > scores[:, :, None]).astype(jnp.int32)
    rank = jnp.sum(gt, axis=2)
    keep = (rank >= rh).astype(xb.dtype)

    wide = jnp.sum(
        jnp.where(seg[None], keep[:, :, None], jnp.float32(0.0)), axis=1
    )
    o_ref[...] = xb * wide[:, :, None]


@jax.jit
def kernel(x):
    b, c, h, w = x.shape
    rh = int(round(0.33 * h))
    xt = jnp.transpose(x, (0, 2, 3, 1)).reshape(b, h * w, c)
    b_blk = 8
    out = pl.pallas_call(
        functools.partial(_topdrop_block, h=h, w=w, rh=rh),
        grid=(b // b_blk,),
        in_specs=[pl.BlockSpec((b_blk, h * w, c), lambda i: (i, 0, 0))],
        out_specs=pl.BlockSpec((b_blk, h * w, c), lambda i: (i, 0, 0)),
        out_shape=jax.ShapeDtypeStruct((b, h * w, c), x.dtype),
    )(xt)
    return jnp.transpose(out.reshape(b, h, w, c), (0, 3, 1, 2))
